# baseline (speedup 1.0000x reference)
import functools

import jax
import jax.numpy as jnp
from jax import lax
from jax.experimental import pallas as pl
from jax.experimental.pallas import tpu as pltpu
from jax.experimental.pallas import tpu_sc as plsc

C = 150
H = 224
W = 224
N = 4
NPIX = N * H * W
HBLK = 56
N_HBLK = H // HBLK

NC = 2
NS = 16
NW = NC * NS
B_PER_W = NPIX // NW
LANES = 16
N_VEC = B_PER_W // LANES
ALPHA_PAD = 160


@functools.partial(
    pl.kernel,
    mesh=plsc.VectorSubcoreMesh(core_axis_name="c", subcore_axis_name="s"),
    out_type=jax.ShapeDtypeStruct((NPIX,), jnp.float32),
    scratch_types=[
        pltpu.VMEM((B_PER_W,), jnp.int32),
        pltpu.VMEM((B_PER_W,), jnp.float32),
        pltpu.VMEM((ALPHA_PAD,), jnp.float32),
    ],
    compiler_params=pltpu.CompilerParams(needs_layout_passes=False),
)
def _sc_alpha(t_hbm, alpha_hbm, a_hbm, t_v, a_v, alpha_v):
    wid = lax.axis_index("s") * NC + lax.axis_index("c")
    base = wid * B_PER_W
    pltpu.sync_copy(t_hbm.at[pl.ds(base, B_PER_W)], t_v)
    pltpu.sync_copy(alpha_hbm, alpha_v)

    def body(i, carry):
        t16 = t_v[pl.ds(i * LANES, LANES)]
        a_v[pl.ds(i * LANES, LANES)] = plsc.load_gather(alpha_v, [t16])
        return carry

    lax.fori_loop(0, N_VEC, body, 0)
    pltpu.sync_copy(a_v, a_hbm.at[pl.ds(base, B_PER_W)])


def _focal_kernel(x_ref, t_ref, a_ref, acc_ref):
    n = pl.program_id(0)
    b = pl.program_id(1)

    @pl.when((n == 0) & (b == 0))
    def _():
        acc_ref[...] = jnp.zeros_like(acc_ref)

    x = x_ref[0]
    t = t_ref[...]
    a = a_ref[0]

    cls = jax.lax.broadcasted_iota(jnp.int32, x.shape, 0)
    xt = jnp.sum(jnp.where(cls == t, x, 0.0), axis=0)

    m = jnp.max(x, axis=0)
    s = jnp.sum(jnp.exp(x - m), axis=0)
    log_pt = xt - m - jnp.log(s)
    pt = jnp.exp(log_pt)
    q = 1.0 - pt
    loss = -a * q * q * log_pt
    acc_ref[...] += jnp.sum(loss).reshape(1, 1)


def kernel(preds, targets, alpha):
    t32 = targets.astype(jnp.int32)
    alpha_pad = jnp.concatenate(
        [alpha.reshape(C), jnp.zeros((ALPHA_PAD - C,), jnp.float32)])
    a_flat = _sc_alpha(t32.reshape(NPIX), alpha_pad)

    acc = pl.pallas_call(
        _focal_kernel,
        grid=(N, N_HBLK),
        in_specs=[
            pl.BlockSpec((1, C, HBLK, W), lambda n, b: (n, 0, b, 0)),
            pl.BlockSpec((1, HBLK, W), lambda n, b: (n, b, 0)),
            pl.BlockSpec((1, HBLK, W), lambda n, b: (n, b, 0)),
        ],
        out_specs=pl.BlockSpec((1, 1), lambda n, b: (0, 0)),
        out_shape=jax.ShapeDtypeStruct((1, 1), jnp.float32),
    )(preds, t32, a_flat.reshape(N, H, W))

    return acc[0, 0] / NPIX
